# Initial kernel scaffold; baseline (speedup 1.0000x reference)
import jax
import jax.numpy as jnp
from jax.experimental import pallas as pl


def kernel(y_true, y_pred):
    raise NotImplementedError("write your pallas kernel here")


# speedup vs baseline: 12.8769x; 12.8769x over previous
import jax
import jax.numpy as jnp
from jax.experimental import pallas as pl
from jax.experimental.pallas import tpu as pltpu

_NEG_POS = 3


def _dense_body(y_ref, x_ref, score_ref, ce_ref):
    x = x_ref[0]
    y = y_ref[0]
    m = jnp.max(x, axis=0)
    e = jnp.exp(x - m[None, :, :])
    s = jnp.sum(e, axis=0)
    lse = m + jnp.log(s)
    cls = jax.lax.broadcasted_iota(jnp.int32, x.shape, 0)
    x_y = jnp.sum(jnp.where(cls == y[None, :, :], x, 0.0), axis=0)
    score_ref[0] = e[0] / s
    ce_ref[0] = lse - x_y


def _select_body(y_ref, score_ref, ce_ref, out_ref, acc_ref):
    b = pl.program_id(0)
    nb = pl.num_programs(0)
    y = y_ref[0]
    sc = score_ref[0]
    ce = ce_ref[0]
    hw = y.shape[0] * y.shape[1]
    pos = y > 0
    num_pos = jnp.sum(pos.astype(jnp.int32))
    k = jnp.minimum(_NEG_POS * num_pos, hw - 1)

    bits = jax.lax.bitcast_convert_type(sc, jnp.int32)

    def step(i, t):
        cand = t | (jnp.int32(1) << (jnp.int32(30) - i))
        cnt = jnp.sum((bits >= cand).astype(jnp.int32))
        return jnp.where(cnt >= k, cand, t)

    t = jax.lax.fori_loop(0, 31, step, jnp.int32(0))

    gt = bits > t
    eq = bits == t
    cnt_gt = jnp.sum(gt.astype(jnp.int32))
    cnt_eq = jnp.sum(eq.astype(jnp.int32))
    m_take = (k - cnt_gt).astype(jnp.float32)
    sum_gt = jnp.sum(jnp.where(gt, ce, 0.0))
    sum_eq = jnp.sum(jnp.where(eq, ce, 0.0))
    neg_sum = sum_gt + sum_eq * m_take / jnp.maximum(cnt_eq.astype(jnp.float32), 1.0)
    pos_sum = jnp.sum(jnp.where(pos, ce, 0.0))

    @pl.when(b == 0)
    def _():
        for i in range(4):
            acc_ref[i] = 0.0

    acc_ref[0] += pos_sum
    acc_ref[1] += num_pos.astype(jnp.float32)
    acc_ref[2] += neg_sum
    acc_ref[3] += k.astype(jnp.float32)

    @pl.when(b == nb - 1)
    def _():
        loss = _NEG_POS * acc_ref[0] / acc_ref[1] + acc_ref[2] / acc_ref[3]
        out_ref[...] = jnp.full(out_ref.shape, loss, jnp.float32)


def kernel(y_true, y_pred):
    bsz, c, h, w = y_pred.shape
    th = 8
    y_true = y_true.astype(jnp.int32)
    score, ce = pl.pallas_call(
        _dense_body,
        grid=(bsz, h // th),
        in_specs=[
            pl.BlockSpec((1, th, w), lambda b, i: (b, i, 0)),
            pl.BlockSpec((1, c, th, w), lambda b, i: (b, 0, i, 0)),
        ],
        out_specs=[
            pl.BlockSpec((1, th, w), lambda b, i: (b, i, 0)),
            pl.BlockSpec((1, th, w), lambda b, i: (b, i, 0)),
        ],
        out_shape=[
            jax.ShapeDtypeStruct((bsz, h, w), jnp.float32),
            jax.ShapeDtypeStruct((bsz, h, w), jnp.float32),
        ],
    )(y_true, y_pred)
    out = pl.pallas_call(
        _select_body,
        grid=(bsz,),
        in_specs=[
            pl.BlockSpec((1, h, w), lambda b: (b, 0, 0)),
            pl.BlockSpec((1, h, w), lambda b: (b, 0, 0)),
            pl.BlockSpec((1, h, w), lambda b: (b, 0, 0)),
        ],
        out_specs=pl.BlockSpec((8, 128), lambda b: (0, 0)),
        out_shape=jax.ShapeDtypeStruct((8, 128), jnp.float32),
        scratch_shapes=[pltpu.SMEM((4,), jnp.float32)],
    )(y_true, score, ce)
    return out[0, 0]
